# baseline (speedup 1.0000x reference)
import functools

import jax
import jax.numpy as jnp
from jax import lax
from jax.experimental import pallas as pl
from jax.experimental.pallas import tpu as pltpu
from jax.experimental.pallas import tpu_sc as plsc

_R, _C = 16384, 200
_TOTAL = _R * _C
_NW = 32
_PER_W = _TOTAL // _NW
_CHUNK = 25600
_NCHUNK = _PER_W // _CHUNK
_UNROLL = 8
_LANES = 16

_ONE_BITS = 0x3F800000


def _sc_body(x_hbm, out_hbm, xin_v, out_v):
    c = lax.axis_index("c")
    s = lax.axis_index("s")
    wid = s * 2 + c
    base = wid * _PER_W

    def chunk_body(k, carry):
        off = base + k * _CHUNK
        pltpu.sync_copy(x_hbm.at[pl.ds(off, _CHUNK)], xin_v)

        def vec_body(i, carry2):
            for u in range(_UNROLL):
                o = (i * _UNROLL + u) * _LANES
                xv = xin_v[pl.ds(o, _LANES)]
                out_v[pl.ds(o, _LANES)] = (1 - (xv << 1)).astype(jnp.float32)
            return carry2

        lax.fori_loop(0, _CHUNK // (_LANES * _UNROLL), vec_body, 0)
        pltpu.sync_copy(out_v, out_hbm.at[pl.ds(off, _CHUNK)])
        return carry

    lax.fori_loop(0, _NCHUNK, chunk_body, 0)


@jax.jit
def kernel(x):
    xf = x.astype(jnp.int32).reshape(_TOTAL)
    mesh = plsc.VectorSubcoreMesh(core_axis_name="c", subcore_axis_name="s")
    f = pl.kernel(
        _sc_body,
        out_type=jax.ShapeDtypeStruct((_TOTAL,), jnp.float32),
        mesh=mesh,
        scratch_types=[
            pltpu.VMEM((_CHUNK,), jnp.int32),
            pltpu.VMEM((_CHUNK,), jnp.float32),
        ],
    )
    out = f(xf)
    return out.reshape(_R, _C)


# speedup vs baseline: 1.0346x; 1.0346x over previous
import jax
import jax.numpy as jnp
from jax import lax
from jax.experimental import pallas as pl
from jax.experimental.pallas import tpu as pltpu
from jax.experimental.pallas import tpu_sc as plsc

_R, _C = 16384, 200
_TOTAL = _R * _C
_NW = 32
_PER_W = _TOTAL // _NW
_CHUNK = 25600
_NCHUNK = _PER_W // _CHUNK
_LANES = 16


def _sc_body(x_hbm, out_hbm, xin0, xin1, outb0, outb1,
             sin0, sin1, sout0, sout1):
    c = lax.axis_index("c")
    s = lax.axis_index("s")
    wid = s * 2 + c
    base = wid * _PER_W

    xin = (xin0, xin1)
    outb = (outb0, outb1)
    sin = (sin0, sin1)
    sout = (sout0, sout1)

    def in_slice(k):
        return x_hbm.at[pl.ds(base + k * _CHUNK, _CHUNK)]

    def out_slice(k):
        return out_hbm.at[pl.ds(base + k * _CHUNK, _CHUNK)]

    pltpu.async_copy(in_slice(0), xin0, sin0)

    for k in range(_NCHUNK):
        b = k % 2
        if k + 1 < _NCHUNK:
            nb = (k + 1) % 2
            pltpu.async_copy(in_slice(k + 1), xin[nb], sin[nb])
        pltpu.make_async_copy(in_slice(k), xin[b], sin[b]).wait()
        if k >= 2:
            pltpu.make_async_copy(outb[b], out_slice(k - 2), sout[b]).wait()

        src = xin[b]
        dst = outb[b]

        @plsc.parallel_loop(0, _CHUNK, step=_LANES, unroll=8)
        def _(i):
            xv = src[pl.ds(i, _LANES)]
            dst[pl.ds(i, _LANES)] = (1 - (xv << 1)).astype(jnp.float32)

        pltpu.async_copy(outb[b], out_slice(k), sout[b])

    for k in range(max(_NCHUNK - 2, 0), _NCHUNK):
        b = k % 2
        pltpu.make_async_copy(outb[b], out_slice(k), sout[b]).wait()


@jax.jit
def kernel(x):
    xf = x.astype(jnp.int32).reshape(_TOTAL)
    mesh = plsc.VectorSubcoreMesh(core_axis_name="c", subcore_axis_name="s")
    f = pl.kernel(
        _sc_body,
        out_type=jax.ShapeDtypeStruct((_TOTAL,), jnp.float32),
        mesh=mesh,
        scratch_types=[
            pltpu.VMEM((_CHUNK,), jnp.int32),
            pltpu.VMEM((_CHUNK,), jnp.int32),
            pltpu.VMEM((_CHUNK,), jnp.float32),
            pltpu.VMEM((_CHUNK,), jnp.float32),
            pltpu.SemaphoreType.DMA,
            pltpu.SemaphoreType.DMA,
            pltpu.SemaphoreType.DMA,
            pltpu.SemaphoreType.DMA,
        ],
    )
    out = f(xf)
    return out.reshape(_R, _C)


# speedup vs baseline: 1.8548x; 1.7928x over previous
import jax
import jax.numpy as jnp
from jax import lax
from jax.experimental import pallas as pl
from jax.experimental.pallas import tpu as pltpu
from jax.experimental.pallas import tpu_sc as plsc

_R, _C = 16384, 200
_NW = 32
_ROWS_W = _R // _NW
_RCH = 128
_NCHUNK = _ROWS_W // _RCH
_LANES = 16
_COLS = tuple(range(0, _C - _LANES + 1, _LANES)) + (_C - _LANES,)


def _sc_body(x_hbm, out_hbm, xin0, xin1, outb0, outb1,
             sin0, sin1, sout0, sout1):
    c = lax.axis_index("c")
    s = lax.axis_index("s")
    wid = s * 2 + c
    base = wid * _ROWS_W

    xin = (xin0, xin1)
    outb = (outb0, outb1)
    sin = (sin0, sin1)
    sout = (sout0, sout1)

    def in_slice(k):
        return x_hbm.at[pl.ds(base + k * _RCH, _RCH), :]

    def out_slice(k):
        return out_hbm.at[pl.ds(base + k * _RCH, _RCH), :]

    pltpu.async_copy(in_slice(0), xin0, sin0)

    for k in range(_NCHUNK):
        b = k % 2
        if k + 1 < _NCHUNK:
            nb = (k + 1) % 2
            pltpu.async_copy(in_slice(k + 1), xin[nb], sin[nb])
        pltpu.make_async_copy(in_slice(k), xin[b], sin[b]).wait()
        if k >= 2:
            pltpu.make_async_copy(outb[b], out_slice(k - 2), sout[b]).wait()

        src = xin[b]
        dst = outb[b]

        @plsc.parallel_loop(0, _RCH, step=1, unroll=2)
        def _(r):
            for col in _COLS:
                xv = src[r, pl.ds(col, _LANES)]
                dst[r, pl.ds(col, _LANES)] = (1 - (xv << 1)).astype(
                    jnp.float32)

        pltpu.async_copy(outb[b], out_slice(k), sout[b])

    for k in range(max(_NCHUNK - 2, 0), _NCHUNK):
        b = k % 2
        pltpu.make_async_copy(outb[b], out_slice(k), sout[b]).wait()


@jax.jit
def kernel(x):
    xi = x.astype(jnp.int32)
    mesh = plsc.VectorSubcoreMesh(core_axis_name="c", subcore_axis_name="s")
    f = pl.kernel(
        _sc_body,
        out_type=jax.ShapeDtypeStruct((_R, _C), jnp.float32),
        mesh=mesh,
        scratch_types=[
            pltpu.VMEM((_RCH, _C), jnp.int32),
            pltpu.VMEM((_RCH, _C), jnp.int32),
            pltpu.VMEM((_RCH, _C), jnp.float32),
            pltpu.VMEM((_RCH, _C), jnp.float32),
            pltpu.SemaphoreType.DMA,
            pltpu.SemaphoreType.DMA,
            pltpu.SemaphoreType.DMA,
            pltpu.SemaphoreType.DMA,
        ],
    )
    return f(xi)


# speedup vs baseline: 3.5534x; 1.9158x over previous
import jax
import jax.numpy as jnp
from jax import lax
from jax.experimental import pallas as pl
from jax.experimental.pallas import tpu as pltpu
from jax.experimental.pallas import tpu_sc as plsc

_R, _C = 200, 16384
_NW = 32
_COLS_W = _C // _NW
_CCH = 128
_NCHUNK = _COLS_W // _CCH
_LANES = 16


def _sc_body(x_hbm, out_hbm, xin0, xin1, outb0, outb1,
             sin0, sin1, sout0, sout1):
    c = lax.axis_index("c")
    s = lax.axis_index("s")
    wid = s * 2 + c
    cbase = wid * _COLS_W

    xin = (xin0, xin1)
    outb = (outb0, outb1)
    sin = (sin0, sin1)
    sout = (sout0, sout1)

    def in_slice(k):
        return x_hbm.at[:, pl.ds(cbase + k * _CCH, _CCH)]

    def out_slice(k):
        return out_hbm.at[:, pl.ds(cbase + k * _CCH, _CCH)]

    pltpu.async_copy(in_slice(0), xin0, sin0)

    for k in range(_NCHUNK):
        b = k % 2
        if k + 1 < _NCHUNK:
            nb = (k + 1) % 2
            pltpu.async_copy(in_slice(k + 1), xin[nb], sin[nb])
        pltpu.make_async_copy(in_slice(k), xin[b], sin[b]).wait()
        if k >= 2:
            pltpu.make_async_copy(outb[b], out_slice(k - 2), sout[b]).wait()

        src = xin[b]
        dst = outb[b]

        @plsc.parallel_loop(0, _R, step=1, unroll=2)
        def _(r):
            for col in range(0, _CCH, _LANES):
                xv = src[r, pl.ds(col, _LANES)]
                dst[r, pl.ds(col, _LANES)] = (1 - (xv << 1)).astype(
                    jnp.float32)

        pltpu.async_copy(outb[b], out_slice(k), sout[b])

    for k in range(max(_NCHUNK - 2, 0), _NCHUNK):
        b = k % 2
        pltpu.make_async_copy(outb[b], out_slice(k), sout[b]).wait()


@jax.jit
def kernel(x):
    xt = x.astype(jnp.int32).T
    mesh = plsc.VectorSubcoreMesh(core_axis_name="c", subcore_axis_name="s")
    f = pl.kernel(
        _sc_body,
        out_type=jax.ShapeDtypeStruct((_R, _C), jnp.float32),
        mesh=mesh,
        scratch_types=[
            pltpu.VMEM((_R, _CCH), jnp.int32),
            pltpu.VMEM((_R, _CCH), jnp.int32),
            pltpu.VMEM((_R, _CCH), jnp.float32),
            pltpu.VMEM((_R, _CCH), jnp.float32),
            pltpu.SemaphoreType.DMA,
            pltpu.SemaphoreType.DMA,
            pltpu.SemaphoreType.DMA,
            pltpu.SemaphoreType.DMA,
        ],
    )
    return f(xt).T


# speedup vs baseline: 3.6245x; 1.0200x over previous
import jax
import jax.numpy as jnp
from jax import lax
from jax.experimental import pallas as pl
from jax.experimental.pallas import tpu as pltpu
from jax.experimental.pallas import tpu_sc as plsc

_R, _C = 200, 16384
_NW = 32
_COLS_W = _C // _NW
_CCH = 128
_NCHUNK = _COLS_W // _CCH
_LANES = 16


def _sc_body(x_hbm, out_hbm, xin0, xin1, outb0, outb1,
             sin0, sin1, sout0, sout1):
    c = lax.axis_index("c")
    s = lax.axis_index("s")
    wid = s * 2 + c
    cbase = wid * _COLS_W

    xin = (xin0, xin1)
    outb = (outb0, outb1)
    sin = (sin0, sin1)
    sout = (sout0, sout1)

    def in_slice(k):
        return x_hbm.at[:, pl.ds(cbase + k * _CCH, _CCH)]

    def out_slice(k):
        return out_hbm.at[:, pl.ds(cbase + k * _CCH, _CCH)]

    pltpu.async_copy(in_slice(0), xin0, sin0)
    pltpu.async_copy(in_slice(1), xin1, sin1)

    def chunk_pair(j, carry):
        for b in range(2):
            k = j * 2 + b
            pltpu.make_async_copy(in_slice(k), xin[b], sin[b]).wait()

            @pl.when(j > 0)
            def _():
                pltpu.make_async_copy(outb[b], out_slice(k - 2),
                                      sout[b]).wait()

            src = xin[b]
            dst = outb[b]

            @plsc.parallel_loop(0, _R, step=1, unroll=2)
            def _(r):
                for col in range(0, _CCH, _LANES):
                    xv = src[r, pl.ds(col, _LANES)]
                    dst[r, pl.ds(col, _LANES)] = (1 - (xv << 1)).astype(
                        jnp.float32)

            pltpu.async_copy(outb[b], out_slice(k), sout[b])

            @pl.when(k + 2 < _NCHUNK)
            def _():
                pltpu.async_copy(in_slice(k + 2), xin[b], sin[b])

        return carry

    lax.fori_loop(0, _NCHUNK // 2, chunk_pair, 0)

    for b in range(2):
        k = _NCHUNK - 2 + b
        pltpu.make_async_copy(outb[b], out_slice(k), sout[b]).wait()


@jax.jit
def kernel(x):
    xt = x.astype(jnp.int32).T
    mesh = plsc.VectorSubcoreMesh(core_axis_name="c", subcore_axis_name="s")
    f = pl.kernel(
        _sc_body,
        out_type=jax.ShapeDtypeStruct((_R, _C), jnp.float32),
        mesh=mesh,
        scratch_types=[
            pltpu.VMEM((_R, _CCH), jnp.int32),
            pltpu.VMEM((_R, _CCH), jnp.int32),
            pltpu.VMEM((_R, _CCH), jnp.float32),
            pltpu.VMEM((_R, _CCH), jnp.float32),
            pltpu.SemaphoreType.DMA,
            pltpu.SemaphoreType.DMA,
            pltpu.SemaphoreType.DMA,
            pltpu.SemaphoreType.DMA,
        ],
    )
    return f(xt).T


# speedup vs baseline: 3.6424x; 1.0049x over previous
import jax
import jax.numpy as jnp
from jax import lax
from jax.experimental import pallas as pl
from jax.experimental.pallas import tpu as pltpu
from jax.experimental.pallas import tpu_sc as plsc

_R, _C = 200, 16384
_NW = 32
_COLS_W = _C // _NW
_CCH = 128
_NCHUNK = _COLS_W // _CCH
_LANES = 16


def _sc_body(x_hbm, out_hbm, xin0, xin1, outb0, outb1,
             sin0, sin1, sout0, sout1):
    c = lax.axis_index("c")
    s = lax.axis_index("s")
    wid = s * 2 + c
    cbase = wid * _COLS_W

    xin = (xin0, xin1)
    outb = (outb0, outb1)
    sin = (sin0, sin1)
    sout = (sout0, sout1)

    def in_slice(k):
        return x_hbm.at[:, pl.ds(cbase + k * _CCH, _CCH)]

    def out_slice(k):
        return out_hbm.at[:, pl.ds(cbase + k * _CCH, _CCH)]

    pltpu.async_copy(in_slice(0), xin0, sin0)
    pltpu.async_copy(in_slice(1), xin1, sin1)

    def chunk_pair(j, carry):
        for b in range(2):
            k = j * 2 + b
            pltpu.make_async_copy(in_slice(k), xin[b], sin[b]).wait()

            @pl.when(j > 0)
            def _():
                pltpu.make_async_copy(outb[b], out_slice(k - 2),
                                      sout[b]).wait()

            src = xin[b]
            dst = outb[b]

            @plsc.parallel_loop(0, _R, step=1, unroll=4)
            def _(r):
                for col in range(0, _CCH, _LANES):
                    xv = src[r, pl.ds(col, _LANES)]
                    dst[r, pl.ds(col, _LANES)] = (1 - (xv << 1)).astype(
                        jnp.float32)

            pltpu.async_copy(outb[b], out_slice(k), sout[b])

            @pl.when(k + 2 < _NCHUNK)
            def _():
                pltpu.async_copy(in_slice(k + 2), xin[b], sin[b])

        return carry

    lax.fori_loop(0, _NCHUNK // 2, chunk_pair, 0)

    for b in range(2):
        k = _NCHUNK - 2 + b
        pltpu.make_async_copy(outb[b], out_slice(k), sout[b]).wait()


@jax.jit
def kernel(x):
    xt = x.astype(jnp.int32).T
    mesh = plsc.VectorSubcoreMesh(core_axis_name="c", subcore_axis_name="s")
    f = pl.kernel(
        _sc_body,
        out_type=jax.ShapeDtypeStruct((_R, _C), jnp.float32),
        mesh=mesh,
        scratch_types=[
            pltpu.VMEM((_R, _CCH), jnp.int32),
            pltpu.VMEM((_R, _CCH), jnp.int32),
            pltpu.VMEM((_R, _CCH), jnp.float32),
            pltpu.VMEM((_R, _CCH), jnp.float32),
            pltpu.SemaphoreType.DMA,
            pltpu.SemaphoreType.DMA,
            pltpu.SemaphoreType.DMA,
            pltpu.SemaphoreType.DMA,
        ],
    )
    return f(xt).T
